# Initial kernel scaffold; baseline (speedup 1.0000x reference)
import jax
import jax.numpy as jnp
from jax.experimental import pallas as pl


def kernel(x, ks_table, pos_table, W1, b1, W2, b2):
    raise NotImplementedError("write your pallas kernel here")


# speedup vs baseline: 2.8273x; 2.8273x over previous
import functools

import jax
import jax.numpy as jnp
from jax import lax
from jax.experimental import pallas as pl
from jax.experimental.pallas import tpu as pltpu
from jax.experimental.pallas import tpu_sc as plsc

EMB = 32
INNER = 128
PACK = 4

NC = 2
NS = 16
NW = NC * NS
CHUNK = 128


def _transform_body(tab_ref, w1_ref, w2_ref, pos_ref, b1_ref, b2_ref,
                    out_ref, pos_out_ref):
    m = jnp.dot(w1_ref[...], w2_ref[...], preferred_element_type=jnp.float32)
    zero = jnp.zeros((EMB, EMB), jnp.float32)
    mblk = jnp.concatenate(
        [jnp.concatenate([m if j == i else zero for j in range(PACK)], axis=1)
         for i in range(PACK)], axis=0)
    out_ref[...] = jnp.dot(tab_ref[...], mblk, preferred_element_type=jnp.float32)

    @pl.when(pl.program_id(0) == 0)
    def _():
        c = jnp.dot(b1_ref[...], w2_ref[...],
                    preferred_element_type=jnp.float32) + b2_ref[...]
        pos_out_ref[...] = jnp.dot(pos_ref[...], m,
                                   preferred_element_type=jnp.float32) + c


def _transform(ks_table, W1, b1, W2, b2, pos_table):
    vocab = ks_table.shape[0]
    t_dim = pos_table.shape[0]
    vrows = vocab // PACK
    blk_rows = 10000
    n_blk = vrows // blk_rows
    tab = ks_table.reshape(vrows, INNER)
    table2, pos2 = pl.pallas_call(
        _transform_body,
        grid=(n_blk,),
        in_specs=[
            pl.BlockSpec((blk_rows, INNER), lambda i: (i, 0)),
            pl.BlockSpec((EMB, INNER), lambda i: (0, 0)),
            pl.BlockSpec((INNER, EMB), lambda i: (0, 0)),
            pl.BlockSpec((t_dim, EMB), lambda i: (0, 0)),
            pl.BlockSpec((1, INNER), lambda i: (0, 0)),
            pl.BlockSpec((1, EMB), lambda i: (0, 0)),
        ],
        out_specs=[
            pl.BlockSpec((blk_rows, INNER), lambda i: (i, 0)),
            pl.BlockSpec((t_dim, EMB), lambda i: (0, 0)),
        ],
        out_shape=[
            jax.ShapeDtypeStruct((vrows, INNER), jnp.float32),
            jax.ShapeDtypeStruct((t_dim, EMB), jnp.float32),
        ],
    )(tab, W1, W2, pos_table, b1.reshape(1, INNER), b2.reshape(1, EMB))
    return table2.reshape(vocab, EMB), pos2


def _sc_gather(table2, xflat, pos2, t_dim):
    rows = xflat.shape[0]
    rows_per_w = rows // NW
    nchunk = rows_per_w // CHUNK
    import math
    pos_rep_rows = math.lcm(CHUNK, t_dim)
    rep = pos_rep_rows // t_dim
    nphase = pos_rep_rows // CHUNK

    mesh = plsc.VectorSubcoreMesh(core_axis_name="c", subcore_axis_name="s")

    @functools.partial(
        pl.kernel,
        mesh=mesh,
        out_type=jax.ShapeDtypeStruct((rows, EMB), jnp.float32),
        scratch_types=[
            pltpu.VMEM((pos_rep_rows, EMB), jnp.float32),
            pltpu.VMEM((CHUNK,), jnp.int32),
            pltpu.VMEM((CHUNK, EMB), jnp.float32),
            pltpu.SemaphoreType.DMA,
        ],
        compiler_params=pltpu.CompilerParams(use_tc_tiling_on_sc=False),
    )
    def body(table2_hbm, x_hbm, pos2_hbm, out_hbm, pos_v, idx_v, rows_v, sem):
        wid = lax.axis_index("s") * NC + lax.axis_index("c")
        base = wid * rows_per_w
        for r in range(rep):
            pltpu.sync_copy(pos2_hbm, pos_v.at[pl.ds(r * t_dim, t_dim)])

        def chunk(i, carry):
            s = base + i * CHUNK
            pltpu.sync_copy(x_hbm.at[pl.ds(s, CHUNK)], idx_v)
            pltpu.async_copy(table2_hbm.at[idx_v], rows_v, sem).wait()
            p = (i % nphase) * CHUNK

            def add_row(j, c2):
                q = p + j
                rows_v[j, pl.ds(0, 16)] = rows_v[j, pl.ds(0, 16)] + pos_v[q, pl.ds(0, 16)]
                rows_v[j, pl.ds(16, 16)] = rows_v[j, pl.ds(16, 16)] + pos_v[q, pl.ds(16, 16)]
                return c2

            lax.fori_loop(0, CHUNK, add_row, 0)
            pltpu.sync_copy(rows_v, out_hbm.at[pl.ds(s, CHUNK)])
            return carry

        lax.fori_loop(0, nchunk, chunk, 0)

    return body(table2, xflat, pos2)


def kernel(x, ks_table, pos_table, W1, b1, W2, b2):
    batch_dim, t_dim = x.shape
    table2, pos2 = _transform(ks_table, W1, b1, W2, b2, pos_table)
    xflat = x.reshape(-1).astype(jnp.int32)
    out = _sc_gather(table2, xflat, pos2, t_dim)
    return out.reshape(batch_dim, t_dim, EMB)


# speedup vs baseline: 4.8215x; 1.7054x over previous
import functools

import jax
import jax.numpy as jnp
from jax import lax
from jax.experimental import pallas as pl
from jax.experimental.pallas import tpu as pltpu
from jax.experimental.pallas import tpu_sc as plsc

EMB = 32
INNER = 128
PACK = 4

NC = 2
NS = 16
NW = NC * NS
HT = 100
NBUF = 4


def _transform_body(tab_ref, w1_ref, w2_ref, pos_ref, b1_ref, b2_ref,
                    out_ref, pos_out_ref):
    m = jnp.dot(w1_ref[...], w2_ref[...], preferred_element_type=jnp.float32)
    zero = jnp.zeros((EMB, EMB), jnp.float32)
    mblk = jnp.concatenate(
        [jnp.concatenate([m if j == i else zero for j in range(PACK)], axis=1)
         for i in range(PACK)], axis=0)
    out_ref[...] = jnp.dot(tab_ref[...], mblk, preferred_element_type=jnp.float32)

    @pl.when(pl.program_id(0) == 0)
    def _():
        c = jnp.dot(b1_ref[...], w2_ref[...],
                    preferred_element_type=jnp.float32) + b2_ref[...]
        pos_out_ref[...] = jnp.dot(pos_ref[...], m,
                                   preferred_element_type=jnp.float32) + c


def _transform(ks_table, W1, b1, W2, b2, pos_table):
    vocab = ks_table.shape[0]
    t_dim = pos_table.shape[0]
    vrows = vocab // PACK
    blk_rows = 10000
    n_blk = vrows // blk_rows
    tab = ks_table.reshape(vrows, INNER)
    table2, pos2 = pl.pallas_call(
        _transform_body,
        grid=(n_blk,),
        in_specs=[
            pl.BlockSpec((blk_rows, INNER), lambda i: (i, 0)),
            pl.BlockSpec((EMB, INNER), lambda i: (0, 0)),
            pl.BlockSpec((INNER, EMB), lambda i: (0, 0)),
            pl.BlockSpec((t_dim, EMB), lambda i: (0, 0)),
            pl.BlockSpec((1, INNER), lambda i: (0, 0)),
            pl.BlockSpec((1, EMB), lambda i: (0, 0)),
        ],
        out_specs=[
            pl.BlockSpec((blk_rows, INNER), lambda i: (i, 0)),
            pl.BlockSpec((t_dim, EMB), lambda i: (0, 0)),
        ],
        out_shape=[
            jax.ShapeDtypeStruct((vrows, INNER), jnp.float32),
            jax.ShapeDtypeStruct((t_dim, EMB), jnp.float32),
        ],
    )(tab, W1, W2, pos_table, b1.reshape(1, INNER), b2.reshape(1, EMB))
    return table2.reshape(vocab, EMB), pos2


def _sc_gather(table2, x2, pos2, batch, t_dim):
    nx = x2.shape[0]
    chunks_per_w = nx // NW
    rows_per_w = chunks_per_w // 2

    mesh = plsc.VectorSubcoreMesh(core_axis_name="c", subcore_axis_name="s")

    @functools.partial(
        pl.kernel,
        mesh=mesh,
        out_type=jax.ShapeDtypeStruct((batch, t_dim, EMB), jnp.float32),
        scratch_types=[
            pltpu.VMEM((t_dim, EMB), jnp.float32),
            pltpu.VMEM((chunks_per_w, HT), jnp.int32),
        ] + [pltpu.VMEM((HT, EMB), jnp.float32) for _ in range(NBUF)]
          + [pltpu.SemaphoreType.DMA for _ in range(2 * NBUF)],
        compiler_params=pltpu.CompilerParams(use_tc_tiling_on_sc=False),
    )
    def body(table2_hbm, x_hbm, pos2_hbm, out_hbm, pos_v, idx_v, *bufs_sems):
        rows = bufs_sems[:NBUF]
        sem_g = bufs_sems[NBUF:2 * NBUF]
        sem_s = bufs_sems[2 * NBUF:]
        wid = lax.axis_index("s") * NC + lax.axis_index("c")
        b0 = wid * rows_per_w

        pltpu.sync_copy(pos2_hbm, pos_v)
        pltpu.sync_copy(x_hbm.at[pl.ds(wid * chunks_per_w, chunks_per_w)], idx_v)

        def fire_gather(j, k):
            pltpu.async_copy(table2_hbm.at[idx_v.at[j]], rows[k], sem_g[k])

        def out_slice(j):
            b = b0 + lax.div(j, 2)
            p = lax.rem(j, 2) * HT
            return out_hbm.at[b, pl.ds(p, HT)]

        fire_gather(0, 0)
        fire_gather(1, 1)

        def group(g, carry):
            for k in range(NBUF):
                j = g * NBUF + k
                kn = (k + 2) % NBUF

                @pl.when(j >= 2)
                def _():
                    pltpu.make_async_copy(rows[kn], out_slice(j - 2),
                                          sem_s[kn]).wait()

                @pl.when(j + 2 < chunks_per_w)
                def _():
                    fire_gather(j + 2, kn)

                pltpu.make_async_copy(table2_hbm.at[idx_v.at[j]], rows[k],
                                      sem_g[k]).wait()
                p = lax.rem(j, 2) * HT

                def add4(i, c2):
                    for k2 in range(4):
                        r = i * 4 + k2
                        for h in (0, 16):
                            rows[k][r, pl.ds(h, 16)] = (
                                rows[k][r, pl.ds(h, 16)]
                                + pos_v[p + r, pl.ds(h, 16)])
                    return c2

                lax.fori_loop(0, HT // 4, add4, 0)
                pltpu.async_copy(rows[k], out_slice(j), sem_s[k])
            return carry

        lax.fori_loop(0, chunks_per_w // NBUF, group, 0)

        for k in (2, 3):
            j = chunks_per_w - NBUF + k
            pltpu.make_async_copy(rows[k], out_slice(j), sem_s[k]).wait()

    return body(table2, x2, pos2)


def kernel(x, ks_table, pos_table, W1, b1, W2, b2):
    batch_dim, t_dim = x.shape
    table2, pos2 = _transform(ks_table, W1, b1, W2, b2, pos_table)
    x2 = x.reshape(batch_dim * t_dim // HT, HT).astype(jnp.int32)
    return _sc_gather(table2, x2, pos2, batch_dim, t_dim)


# speedup vs baseline: 10.4164x; 2.1604x over previous
import functools

import jax
import jax.numpy as jnp
from jax import lax
from jax.experimental import pallas as pl
from jax.experimental.pallas import tpu as pltpu
from jax.experimental.pallas import tpu_sc as plsc

EMB = 32
INNER = 128

NC = 2
NS = 16
NW = NC * NS
BT_PER_W = 4
BW = BT_PER_W * 128


def _sc_gather(table, xt3, batch, t_dim):
    mesh = plsc.VectorSubcoreMesh(core_axis_name="c", subcore_axis_name="s")

    @functools.partial(
        pl.kernel,
        mesh=mesh,
        out_type=jax.ShapeDtypeStruct((t_dim, batch * EMB // 128, 128),
                                      jnp.float32),
        scratch_types=[pltpu.VMEM((BT_PER_W, 128), jnp.int32) for _ in range(2)]
          + [pltpu.VMEM((BW, EMB), jnp.float32) for _ in range(2)]
          + [pltpu.SemaphoreType.DMA for _ in range(6)],
        compiler_params=pltpu.CompilerParams(use_tc_tiling_on_sc=False),
    )
    def body(tab_hbm, xt_hbm, out_hbm,
             idx0, idx1, rows0, rows1, si0, si1, sg0, sg1, ss0, ss1):
        idx = (idx0, idx1)
        rows = (rows0, rows1)
        sem_i = (si0, si1)
        sem_g = (sg0, sg1)
        sem_s = (ss0, ss1)
        wid = lax.axis_index("s") * NC + lax.axis_index("c")
        bt0 = wid * BT_PER_W
        r0 = wid * 128

        def fire_idx(t, p):
            pltpu.async_copy(xt_hbm.at[t, pl.ds(bt0, BT_PER_W)], idx[p],
                             sem_i[p])

        def wait_idx(p):
            pltpu.make_async_copy(xt_hbm.at[0, pl.ds(0, BT_PER_W)], idx[p],
                                  sem_i[p]).wait()

        def fire_gathers(p):
            for q in range(BT_PER_W):
                pltpu.async_copy(tab_hbm.at[idx[p].at[q]],
                                 rows[p].at[pl.ds(q * 128, 128)], sem_g[p])

        def wait_gathers(p):
            pltpu.make_async_copy(tab_hbm.at[pl.ds(0, BW)], rows[p],
                                  sem_g[p]).wait()

        def fire_store(t, p):
            for q in range(BT_PER_W):
                pltpu.async_copy(rows[p].at[pl.ds(q * 128, 128)],
                                 out_hbm.at[t, pl.ds(r0, 128),
                                            pl.ds(q * EMB, EMB)], sem_s[p])

        def wait_store(p):
            for q in range(BT_PER_W):
                pltpu.make_async_copy(rows[p].at[pl.ds(q * 128, 128)],
                                      out_hbm.at[0, pl.ds(0, 128),
                                                 pl.ds(0, EMB)],
                                      sem_s[p]).wait()

        def t_iter(t, p):
            q = 1 - p

            @pl.when(t + 1 < t_dim)
            def _():
                wait_idx(q)

                @pl.when(t >= 1)
                def _():
                    wait_store(q)

                fire_gathers(q)

            wait_gathers(p)

            @pl.when(t + 2 < t_dim)
            def _():
                fire_idx(t + 2, p)

            fire_store(t, p)

        pltpu.sync_copy(xt_hbm.at[0, pl.ds(bt0, BT_PER_W)], idx[0])
        fire_gathers(0)
        fire_idx(1, 1)

        def pair(m, carry):
            t_iter(2 * m, 0)
            t_iter(2 * m + 1, 1)
            return carry

        lax.fori_loop(0, t_dim // 2, pair, 0)
        wait_store(0)
        wait_store(1)

    return body(table, xt3)


def _finish_body(tmp_ref, w1_ref, w2_ref, pos_ref, b1_ref, b2_ref, out_ref):
    f32 = jnp.float32
    m = jnp.dot(w1_ref[...], w2_ref[...], preferred_element_type=f32)
    ptc = lax.dot_general(m, pos_ref[0], (((0,), (1,)), ((), ())),
                          preferred_element_type=f32)
    ccol = lax.dot_general(w2_ref[...], b1_ref[...], (((0,), (1,)), ((), ())),
                           preferred_element_type=f32)
    col = ptc + ccol + b2_ref[...]
    v = tmp_ref[0]
    qn = v.shape[0]
    for a in range(4):
        va = v[:, 32 * a:32 * (a + 1)]
        ga = lax.dot_general(m, va, (((0,), (1,)), ((), ())),
                             preferred_element_type=f32)
        out_ref[0, :, a * qn:(a + 1) * qn] = ga + col


def _finish(temp4, W1, b1, W2, b2, pos_table, batch, t_dim):
    pos3 = pos_table.reshape(t_dim, 1, EMB)
    out3 = pl.pallas_call(
        _finish_body,
        grid=(t_dim,),
        in_specs=[
            pl.BlockSpec((1, batch * EMB // 128, 128), lambda t: (t, 0, 0)),
            pl.BlockSpec((EMB, INNER), lambda t: (0, 0)),
            pl.BlockSpec((INNER, EMB), lambda t: (0, 0)),
            pl.BlockSpec((1, 1, EMB), lambda t: (t, 0, 0)),
            pl.BlockSpec((1, INNER), lambda t: (0, 0)),
            pl.BlockSpec((EMB, 1), lambda t: (0, 0)),
        ],
        out_specs=pl.BlockSpec((1, EMB, batch), lambda t: (t, 0, 0)),
        out_shape=jax.ShapeDtypeStruct((t_dim, EMB, batch), jnp.float32),
    )(temp4, W1, W2, pos3, b1.reshape(1, INNER), b2.reshape(EMB, 1))
    return out3


def kernel(x, ks_table, pos_table, W1, b1, W2, b2):
    batch_dim, t_dim = x.shape
    xt = jnp.transpose(x).astype(jnp.int32)
    xt_perm = jnp.transpose(
        xt.reshape(t_dim, BT_PER_W, NW, 128), (0, 2, 1, 3))
    xt3 = xt_perm.reshape(t_dim, batch_dim // 128, 128)
    temp4 = _sc_gather(ks_table, xt3, batch_dim, t_dim)
    out3 = _finish(temp4, W1, b1, W2, b2, pos_table, batch_dim, t_dim)
    out = jnp.transpose(out3, (2, 0, 1))
    return out


# speedup vs baseline: 10.8512x; 1.0417x over previous
import functools

import jax
import jax.numpy as jnp
from jax import lax
from jax.experimental import pallas as pl
from jax.experimental.pallas import tpu as pltpu
from jax.experimental.pallas import tpu_sc as plsc

EMB = 32
INNER = 128

NC = 2
NS = 16
NW = NC * NS
BT_PER_W = 4
BW = BT_PER_W * 128


def _sc_gather(table, xt3, batch, t_dim):
    mesh = plsc.VectorSubcoreMesh(core_axis_name="c", subcore_axis_name="s")

    @functools.partial(
        pl.kernel,
        mesh=mesh,
        out_type=jax.ShapeDtypeStruct((t_dim, batch * EMB // 128, 128),
                                      jnp.float32),
        scratch_types=[pltpu.VMEM((BT_PER_W, 128), jnp.int32) for _ in range(2)]
          + [pltpu.VMEM((BW, EMB), jnp.float32) for _ in range(2)]
          + [pltpu.SemaphoreType.DMA for _ in range(6)],
        compiler_params=pltpu.CompilerParams(use_tc_tiling_on_sc=False),
    )
    def body(tab_hbm, xt_hbm, out_hbm,
             idx0, idx1, rows0, rows1, si0, si1, sg0, sg1, ss0, ss1):
        idx = (idx0, idx1)
        rows = (rows0, rows1)
        sem_i = (si0, si1)
        sem_g = (sg0, sg1)
        sem_s = (ss0, ss1)
        wid = lax.axis_index("s") * NC + lax.axis_index("c")
        bt0 = wid * BT_PER_W
        r0 = wid * 128

        def fire_idx(t, p):
            pltpu.async_copy(xt_hbm.at[t, pl.ds(bt0, BT_PER_W)], idx[p],
                             sem_i[p])

        def wait_idx(p):
            pltpu.make_async_copy(xt_hbm.at[0, pl.ds(0, BT_PER_W)], idx[p],
                                  sem_i[p]).wait()

        def fire_gathers(p):
            for q in range(BT_PER_W):
                pltpu.async_copy(tab_hbm.at[idx[p].at[q]],
                                 rows[p].at[pl.ds(q * 128, 128)], sem_g[p])

        def wait_gathers(p):
            pltpu.make_async_copy(tab_hbm.at[pl.ds(0, BW)], rows[p],
                                  sem_g[p]).wait()

        def fire_store(t, p):
            for q in range(BT_PER_W):
                pltpu.async_copy(rows[p].at[pl.ds(q * 128, 128)],
                                 out_hbm.at[t, pl.ds(r0, 128),
                                            pl.ds(q * EMB, EMB)], sem_s[p])

        def wait_store(p):
            for q in range(BT_PER_W):
                pltpu.make_async_copy(rows[p].at[pl.ds(q * 128, 128)],
                                      out_hbm.at[0, pl.ds(0, 128),
                                                 pl.ds(0, EMB)],
                                      sem_s[p]).wait()

        def t_iter(t, p):
            q = 1 - p

            @pl.when(t + 1 < t_dim)
            def _():
                wait_idx(q)

                @pl.when(t >= 1)
                def _():
                    wait_store(q)

                fire_gathers(q)

            wait_gathers(p)

            @pl.when(t + 2 < t_dim)
            def _():
                fire_idx(t + 2, p)

            fire_store(t, p)

        pltpu.sync_copy(xt_hbm.at[0, pl.ds(bt0, BT_PER_W)], idx[0])
        fire_gathers(0)
        fire_idx(1, 1)

        def pair(m, carry):
            t_iter(2 * m, 0)
            t_iter(2 * m + 1, 1)
            return carry

        lax.fori_loop(0, t_dim // 2, pair, 0)
        wait_store(0)
        wait_store(1)

    return body(table, xt3)


def _finish_body(tmp_ref, w1_ref, w2_ref, pos_ref, b1_ref, b2_ref, out_ref):
    f32 = jnp.float32
    _finish_common(tmp_ref, w1_ref, w2_ref, pos_ref, b1_ref, b2_ref, out_ref)


def _finish_body_alias(tmp_ref, w1_ref, w2_ref, pos_ref, b1_ref, b2_ref,
                       alias_ref, out_ref):
    del alias_ref
    _finish_common(tmp_ref, w1_ref, w2_ref, pos_ref, b1_ref, b2_ref, out_ref)


def _finish_common(tmp_ref, w1_ref, w2_ref, pos_ref, b1_ref, b2_ref, out_ref):
    f32 = jnp.float32
    m = jnp.dot(w1_ref[...], w2_ref[...], preferred_element_type=f32)
    ptc = lax.dot_general(m, pos_ref[0], (((0,), (1,)), ((), ())),
                          preferred_element_type=f32)
    ccol = lax.dot_general(w2_ref[...], b1_ref[...], (((0,), (1,)), ((), ())),
                           preferred_element_type=f32)
    col = ptc + ccol + b2_ref[...]
    v = tmp_ref[0]
    qn = v.shape[0]
    for a in range(4):
        va = v[:, 32 * a:32 * (a + 1)]
        ga = lax.dot_general(m, va, (((0,), (1,)), ((), ())),
                             preferred_element_type=f32)
        out_ref[0, :, a * qn:(a + 1) * qn] = ga + col


def _finish(temp4, W1, b1, W2, b2, pos3, batch, total_t, t_off, alias=None):
    t_half = temp4.shape[0]
    in_specs = [
        pl.BlockSpec((1, batch * EMB // 128, 128), lambda t: (t, 0, 0)),
        pl.BlockSpec((EMB, INNER), lambda t: (0, 0)),
        pl.BlockSpec((INNER, EMB), lambda t: (0, 0)),
        pl.BlockSpec((1, 1, EMB), lambda t: (t, 0, 0)),
        pl.BlockSpec((1, INNER), lambda t: (0, 0)),
        pl.BlockSpec((EMB, 1), lambda t: (0, 0)),
    ]
    args = [temp4, W1, W2, pos3, b1.reshape(1, INNER), b2.reshape(EMB, 1)]
    kwargs = {}
    body = _finish_body
    if alias is not None:
        in_specs.append(pl.BlockSpec(memory_space=pl.ANY))
        args.append(alias)
        kwargs["input_output_aliases"] = {6: 0}
        body = _finish_body_alias
    out3 = pl.pallas_call(
        body,
        grid=(t_half,),
        in_specs=in_specs,
        out_specs=pl.BlockSpec((1, EMB, batch),
                               lambda t, o=t_off: (t + o, 0, 0)),
        out_shape=jax.ShapeDtypeStruct((total_t, EMB, batch), jnp.float32),
        **kwargs,
    )(*args)
    return out3


def kernel(x, ks_table, pos_table, W1, b1, W2, b2):
    batch_dim, t_dim = x.shape
    xt = jnp.transpose(x).astype(jnp.int32)
    xt_perm = jnp.transpose(
        xt.reshape(t_dim, BT_PER_W, NW, 128), (0, 2, 1, 3))
    xt3 = xt_perm.reshape(t_dim, batch_dim // 128, 128)
    th = t_dim // 2
    pos3 = pos_table.reshape(t_dim, 1, EMB)
    temp_a = _sc_gather(ks_table, xt3[:th], batch_dim, th)
    temp_b = _sc_gather(ks_table, xt3[th:], batch_dim, t_dim - th)
    out_a = _finish(temp_a, W1, b1, W2, b2, pos3[:th], batch_dim, t_dim, 0)
    out3 = _finish(temp_b, W1, b1, W2, b2, pos3[th:], batch_dim, t_dim, th,
                   alias=out_a)
    out = jnp.transpose(out3, (2, 0, 1))
    return out
